# baseline (speedup 1.0000x reference)
import functools

import jax
import jax.numpy as jnp
from jax import lax
from jax.experimental import pallas as pl
from jax.experimental.pallas import tpu as pltpu
from jax.experimental.pallas import tpu_sc as plsc

N = 10000
E = 320000
D = 128
DE = 16

NC = 2
NS = 16
NW = NC * NS
EPW = E // NW
CHUNK = 80
NCHUNKS = EPW // CHUNK
RPT = 624
TAIL = N - NS * RPT
VPR = D // 16
NBUF = 2
NIBUF = NBUF + 1


def _sc_edge_agg(hW, eW, snd, rcv):
    mesh = plsc.VectorSubcoreMesh(core_axis_name="c", subcore_axis_name="s")

    @functools.partial(
        pl.kernel,
        out_type=jax.ShapeDtypeStruct((NC, N, D), jnp.float32),
        mesh=mesh,
        scratch_types=[
            pltpu.VMEM((NIBUF, CHUNK), jnp.int32),
            pltpu.VMEM((NIBUF, CHUNK), jnp.int32),
            pltpu.VMEM((NBUF, CHUNK, D), jnp.float32),
            pltpu.VMEM((NBUF, CHUNK, D), jnp.float32),
            pltpu.VMEM_SHARED((N, D), jnp.float32),
            pltpu.SemaphoreType.DMA((NIBUF,)),
            pltpu.SemaphoreType.DMA((NIBUF,)),
            pltpu.SemaphoreType.DMA((NBUF,)),
            pltpu.SemaphoreType.DMA((NBUF,)),
            pltpu.SemaphoreType.DMA((NBUF,)),
        ],
    )
    def k(hW_hbm, eW_hbm, snd_hbm, rcv_hbm, out_hbm,
          sidx, ridx, grows, erows, agg, sem_si, sem_ri, sem_g, sem_e, sem_s):
        c = lax.axis_index("c")
        s = lax.axis_index("s")
        wid = c * NS + s
        base0 = wid * EPW

        zero = jnp.zeros((16,), jnp.float32)

        def zset(i, carry):
            for w in range(VPR):
                grows[0, i, pl.ds(w * 16, 16)] = zero
            return carry

        lax.fori_loop(0, CHUNK, zset, 0)
        for j in range(RPT // CHUNK):
            pltpu.sync_copy(grows.at[0],
                            agg.at[pl.ds(s * RPT + j * CHUNK, CHUNK)])
        rem = RPT % CHUNK
        if rem:
            pltpu.sync_copy(grows.at[0, pl.ds(0, rem)],
                            agg.at[pl.ds(s * RPT + RPT - rem, rem)])

        @pl.when(s == NS - 1)
        def _():
            pltpu.sync_copy(grows.at[0, pl.ds(0, TAIL)],
                            agg.at[pl.ds(NS * RPT, TAIL)])

        plsc.subcore_barrier()

        def issue_idx(ci, islot):
            off = pl.ds(base0 + ci * CHUNK, CHUNK)
            pltpu.async_copy(snd_hbm.at[off], sidx.at[islot], sem_si.at[islot])
            pltpu.async_copy(rcv_hbm.at[off], ridx.at[islot], sem_ri.at[islot])

        def issue_loads(ci, slot, islot):
            pltpu.async_copy(hW_hbm.at[sidx.at[islot]], grows.at[slot],
                             sem_g.at[slot])
            pltpu.async_copy(eW_hbm.at[pl.ds(base0 + ci * CHUNK, CHUNK)],
                             erows.at[slot], sem_e.at[slot])

        def drain_rows(dst, sem):
            pltpu.make_async_copy(eW_hbm.at[pl.ds(0, CHUNK)], dst, sem).wait()

        def drain_idx(dst, sem):
            pltpu.make_async_copy(snd_hbm.at[pl.ds(0, CHUNK)], dst, sem).wait()

        issue_idx(0, 0)
        issue_idx(1, 1)
        drain_idx(sidx.at[0], sem_si.at[0])
        drain_idx(ridx.at[0], sem_ri.at[0])
        issue_loads(0, 0, 0)

        def chunk_body(ci, carry):
            slot = lax.rem(ci, NBUF)
            nslot = lax.rem(ci + 1, NBUF)
            islot = lax.rem(ci, NIBUF)
            nislot = lax.rem(ci + 1, NIBUF)
            n2islot = lax.rem(ci + 2, NIBUF)

            @pl.when(ci >= NBUF - 1)
            def _():
                drain_rows(grows.at[nslot], sem_s.at[nslot])

            @pl.when(ci + 1 < NCHUNKS)
            def _():
                drain_idx(sidx.at[nislot], sem_si.at[nislot])
                drain_idx(ridx.at[nislot], sem_ri.at[nislot])
                issue_loads(ci + 1, nslot, nislot)

            @pl.when(ci + 2 < NCHUNKS)
            def _():
                issue_idx(ci + 2, n2islot)

            drain_rows(grows.at[slot], sem_g.at[slot])
            drain_rows(erows.at[slot], sem_e.at[slot])

            def ebody(e, ecarry):
                for w in range(VPR):
                    sl = pl.ds(w * 16, 16)
                    grows[slot, e, sl] = jnp.maximum(
                        grows[slot, e, sl] + erows[slot, e, sl], 0.0)
                return ecarry

            lax.fori_loop(0, CHUNK, ebody, 0)
            pltpu.async_copy(grows.at[slot], agg.at[ridx.at[islot]],
                             sem_s.at[slot], add=True)
            return carry

        lax.fori_loop(0, NCHUNKS, chunk_body, 0)
        for cj in range(NCHUNKS - NBUF + 1, NCHUNKS):
            drain_rows(grows.at[cj % NBUF], sem_s.at[cj % NBUF])
        plsc.subcore_barrier()

        pltpu.sync_copy(agg.at[pl.ds(s * RPT, RPT)],
                        out_hbm.at[c, pl.ds(s * RPT, RPT)])

        @pl.when(s == NS - 1)
        def _():
            pltpu.sync_copy(agg.at[pl.ds(NS * RPT, TAIL)],
                            out_hbm.at[c, pl.ds(NS * RPT, TAIL)])

    return k(hW, eW, snd, rcv)


_NBLK = 1000
_EBLK = 4000


def _node_proj_body(h_ref, w_ref, b_ref, o_ref):
    o_ref[...] = jnp.dot(h_ref[...], w_ref[...],
                         preferred_element_type=jnp.float32) + b_ref[...]


def _node_proj(h, w, b):
    return pl.pallas_call(
        _node_proj_body,
        grid=(N // _NBLK,),
        in_specs=[
            pl.BlockSpec((_NBLK, D), lambda i: (i, 0)),
            pl.BlockSpec((D, D), lambda i: (0, 0)),
            pl.BlockSpec((1, D), lambda i: (0, 0)),
        ],
        out_specs=pl.BlockSpec((_NBLK, D), lambda i: (i, 0)),
        out_shape=jax.ShapeDtypeStruct((N, D), jnp.float32),
    )(h, w, b.reshape(1, D))


def _edge_proj_body(a_ref, w_ref, o_ref):
    o_ref[...] = jnp.dot(a_ref[...], w_ref[...],
                         preferred_element_type=jnp.float32)


def _edge_proj(ea, w):
    return pl.pallas_call(
        _edge_proj_body,
        grid=(E // _EBLK,),
        in_specs=[
            pl.BlockSpec((_EBLK, DE), lambda i: (i, 0)),
            pl.BlockSpec((DE, D), lambda i: (0, 0)),
        ],
        out_specs=pl.BlockSpec((_EBLK, D), lambda i: (i, 0)),
        out_shape=jax.ShapeDtypeStruct((E, D), jnp.float32),
    )(ea, w)


def _update_body(p_ref, h_ref, wu_ref, ws_ref, b_ref, o_ref):
    agg = p_ref[0] + p_ref[1]
    t = (jnp.dot(agg, wu_ref[...], preferred_element_type=jnp.float32)
         + jnp.dot(h_ref[...], ws_ref[...], preferred_element_type=jnp.float32)
         + b_ref[...])
    o_ref[...] = h_ref[...] + jnp.maximum(t, 0.0)


def _update(parts, h, wu, ws, b):
    return pl.pallas_call(
        _update_body,
        grid=(N // _NBLK,),
        in_specs=[
            pl.BlockSpec((NC, _NBLK, D), lambda i: (0, i, 0)),
            pl.BlockSpec((_NBLK, D), lambda i: (i, 0)),
            pl.BlockSpec((D, D), lambda i: (0, 0)),
            pl.BlockSpec((D, D), lambda i: (0, 0)),
            pl.BlockSpec((1, D), lambda i: (0, 0)),
        ],
        out_specs=pl.BlockSpec((_NBLK, D), lambda i: (i, 0)),
        out_shape=jax.ShapeDtypeStruct((N, D), jnp.float32),
    )(parts, h, wu, ws, b.reshape(1, D))


def kernel(x, edge_index, edge_attr, W_msg, W_edge, W_upd, W_self, b_msg, b_upd):
    snd = edge_index[0]
    rcv = edge_index[1]
    h = x
    n_layers = W_msg.shape[0]
    for l in range(n_layers):
        eW = _edge_proj(edge_attr, W_edge[l])
        hW = _node_proj(h, W_msg[l], b_msg[l])
        parts = _sc_edge_agg(hW, eW, snd, rcv)
        h = _update(parts, h, W_upd[l], W_self[l], b_upd[l])
    return h


# speedup vs baseline: 1.5434x; 1.5434x over previous
import functools

import jax
import jax.numpy as jnp
from jax import lax
from jax.experimental import pallas as pl
from jax.experimental.pallas import tpu as pltpu
from jax.experimental.pallas import tpu_sc as plsc

N = 10000
E = 320000
D = 128
DE = 16

NC = 2
NS = 16
NW = NC * NS
EPW = E // NW
CHUNK = 128
NFULL = EPW // CHUNK
TAILE = EPW - NFULL * CHUNK
RPT = 624
TAIL = N - NS * RPT
VPR = D // 16


def _sc_edge_agg(hW, eW, snd, rcv):
    mesh = plsc.VectorSubcoreMesh(core_axis_name="c", subcore_axis_name="s")

    @functools.partial(
        pl.kernel,
        out_type=jax.ShapeDtypeStruct((NC, N, D), jnp.float32),
        mesh=mesh,
        scratch_types=[
            pltpu.VMEM((2, CHUNK), jnp.int32),
            pltpu.VMEM((2, CHUNK), jnp.int32),
            pltpu.VMEM((TAILE,), jnp.int32),
            pltpu.VMEM((TAILE,), jnp.int32),
            pltpu.VMEM((CHUNK, D), jnp.float32),
            pltpu.VMEM((CHUNK, D), jnp.float32),
            pltpu.VMEM_SHARED((N, D), jnp.float32),
            pltpu.SemaphoreType.DMA((2,)),
            pltpu.SemaphoreType.DMA((2,)),
            pltpu.SemaphoreType.DMA,
            pltpu.SemaphoreType.DMA,
        ],
    )
    def k(hW_hbm, eW_hbm, snd_hbm, rcv_hbm, out_hbm,
          sidx, ridx, tsidx, tridx, grows, erows, agg, sem_si, sem_ri,
          sem_g, sem_e):
        c = lax.axis_index("c")
        s = lax.axis_index("s")
        wid = c * NS + s
        base0 = wid * EPW

        zero = jnp.zeros((16,), jnp.float32)

        def zset(i, carry):
            for w in range(VPR):
                grows[i, pl.ds(w * 16, 16)] = zero
            return carry

        lax.fori_loop(0, CHUNK, zset, 0)
        for j in range(RPT // CHUNK):
            pltpu.sync_copy(grows,
                            agg.at[pl.ds(s * RPT + j * CHUNK, CHUNK)])
        rem = RPT % CHUNK
        if rem:
            pltpu.sync_copy(grows.at[pl.ds(0, rem)],
                            agg.at[pl.ds(s * RPT + RPT - rem, rem)])

        @pl.when(s == NS - 1)
        def _():
            pltpu.sync_copy(grows.at[pl.ds(0, TAIL)],
                            agg.at[pl.ds(NS * RPT, TAIL)])

        plsc.subcore_barrier()

        def issue_idx(ci, islot):
            off = pl.ds(base0 + ci * CHUNK, CHUNK)
            pltpu.async_copy(snd_hbm.at[off], sidx.at[islot], sem_si.at[islot])
            pltpu.async_copy(rcv_hbm.at[off], ridx.at[islot], sem_ri.at[islot])

        def drain_idx(dst, sem):
            pltpu.make_async_copy(snd_hbm.at[pl.ds(0, CHUNK)], dst, sem).wait()

        issue_idx(0, 0)

        def chunk_body(ci, carry):
            b = lax.rem(ci, 2)
            nb = lax.rem(ci + 1, 2)
            drain_idx(sidx.at[b], sem_si.at[b])
            drain_idx(ridx.at[b], sem_ri.at[b])
            cg = pltpu.async_copy(hW_hbm.at[sidx.at[b]], grows, sem_g)
            ce = pltpu.async_copy(
                eW_hbm.at[pl.ds(base0 + ci * CHUNK, CHUNK)], erows, sem_e)

            @pl.when(ci + 1 < NFULL)
            def _():
                issue_idx(ci + 1, nb)

            ce.wait()
            cg.wait()

            def ebody(e, ecarry):
                for w in range(VPR):
                    sl = pl.ds(w * 16, 16)
                    grows[e, sl] = jnp.maximum(
                        grows[e, sl] + erows[e, sl], 0.0)
                return ecarry

            lax.fori_loop(0, CHUNK, ebody, 0)
            pltpu.sync_copy(grows, agg.at[ridx.at[b]], add=True)
            return carry

        lax.fori_loop(0, NFULL, chunk_body, 0)

        toff = pl.ds(base0 + NFULL * CHUNK, TAILE)
        pltpu.sync_copy(snd_hbm.at[toff], tsidx)
        pltpu.sync_copy(rcv_hbm.at[toff], tridx)
        cg = pltpu.async_copy(hW_hbm.at[tsidx], grows.at[pl.ds(0, TAILE)],
                              sem_g)
        ce = pltpu.async_copy(eW_hbm.at[toff], erows.at[pl.ds(0, TAILE)],
                              sem_e)
        ce.wait()
        cg.wait()

        def tbody(e, ecarry):
            for w in range(VPR):
                sl = pl.ds(w * 16, 16)
                grows[e, sl] = jnp.maximum(grows[e, sl] + erows[e, sl], 0.0)
            return ecarry

        lax.fori_loop(0, TAILE, tbody, 0)
        pltpu.sync_copy(grows.at[pl.ds(0, TAILE)], agg.at[tridx], add=True)

        plsc.subcore_barrier()

        pltpu.sync_copy(agg.at[pl.ds(s * RPT, RPT)],
                        out_hbm.at[c, pl.ds(s * RPT, RPT)])

        @pl.when(s == NS - 1)
        def _():
            pltpu.sync_copy(agg.at[pl.ds(NS * RPT, TAIL)],
                            out_hbm.at[c, pl.ds(NS * RPT, TAIL)])

    return k(hW, eW, snd, rcv)


_NBLK = 1000
_EBLK = 4000


def _node_proj_body(h_ref, w_ref, b_ref, o_ref):
    o_ref[...] = jnp.dot(h_ref[...], w_ref[...],
                         preferred_element_type=jnp.float32) + b_ref[...]


def _node_proj(h, w, b):
    return pl.pallas_call(
        _node_proj_body,
        grid=(N // _NBLK,),
        in_specs=[
            pl.BlockSpec((_NBLK, D), lambda i: (i, 0)),
            pl.BlockSpec((D, D), lambda i: (0, 0)),
            pl.BlockSpec((1, D), lambda i: (0, 0)),
        ],
        out_specs=pl.BlockSpec((_NBLK, D), lambda i: (i, 0)),
        out_shape=jax.ShapeDtypeStruct((N, D), jnp.float32),
    )(h, w, b.reshape(1, D))


def _edge_proj_body(a_ref, w_ref, o_ref):
    o_ref[...] = jnp.dot(a_ref[...], w_ref[...],
                         preferred_element_type=jnp.float32)


def _edge_proj(ea, w):
    return pl.pallas_call(
        _edge_proj_body,
        grid=(E // _EBLK,),
        in_specs=[
            pl.BlockSpec((_EBLK, DE), lambda i: (i, 0)),
            pl.BlockSpec((DE, D), lambda i: (0, 0)),
        ],
        out_specs=pl.BlockSpec((_EBLK, D), lambda i: (i, 0)),
        out_shape=jax.ShapeDtypeStruct((E, D), jnp.float32),
    )(ea, w)


def _update_body(p_ref, h_ref, wu_ref, ws_ref, b_ref, o_ref):
    agg = p_ref[0] + p_ref[1]
    t = (jnp.dot(agg, wu_ref[...], preferred_element_type=jnp.float32)
         + jnp.dot(h_ref[...], ws_ref[...], preferred_element_type=jnp.float32)
         + b_ref[...])
    o_ref[...] = h_ref[...] + jnp.maximum(t, 0.0)


def _update(parts, h, wu, ws, b):
    return pl.pallas_call(
        _update_body,
        grid=(N // _NBLK,),
        in_specs=[
            pl.BlockSpec((NC, _NBLK, D), lambda i: (0, i, 0)),
            pl.BlockSpec((_NBLK, D), lambda i: (i, 0)),
            pl.BlockSpec((D, D), lambda i: (0, 0)),
            pl.BlockSpec((D, D), lambda i: (0, 0)),
            pl.BlockSpec((1, D), lambda i: (0, 0)),
        ],
        out_specs=pl.BlockSpec((_NBLK, D), lambda i: (i, 0)),
        out_shape=jax.ShapeDtypeStruct((N, D), jnp.float32),
    )(parts, h, wu, ws, b.reshape(1, D))


def kernel(x, edge_index, edge_attr, W_msg, W_edge, W_upd, W_self, b_msg, b_upd):
    snd = edge_index[0]
    rcv = edge_index[1]
    h = x
    n_layers = W_msg.shape[0]
    eWs = [_edge_proj(edge_attr, W_edge[l]) for l in range(n_layers)]
    for l in range(n_layers):
        hW = _node_proj(h, W_msg[l], b_msg[l])
        parts = _sc_edge_agg(hW, eWs[l], snd, rcv)
        h = _update(parts, h, W_upd[l], W_self[l], b_upd[l])
    return h
